# baseline (speedup 1.0000x reference)
import functools

import jax
import jax.numpy as jnp
from jax import lax
from jax.experimental import pallas as pl
from jax.experimental.pallas import tpu as pltpu
from jax.experimental.pallas import tpu_sc as plsc

LATENT = 64
CODES = 1024
B = 16384
ROWS_PER_BLOCK = 1024

NC = 2
NS = 16
NW = NC * NS
BPW = B // NW
CHUNK = 128
NCHUNK = BPW // CHUNK


def _argmin_body(x_ref, e_ref, idx_ref):
    xb = x_ref[...]
    em = e_ref[...]
    sim = jnp.dot(xb, em, preferred_element_type=jnp.float32)
    x_sq = jnp.sum(xb * xb, axis=1, keepdims=True)
    e_sq = jnp.sum(em * em, axis=0, keepdims=True)
    dist = (x_sq + e_sq) - 2.0 * sim
    minval = jnp.min(dist, axis=1, keepdims=True)
    cols = lax.broadcasted_iota(jnp.int32, dist.shape, 1)
    idx_ref[...] = jnp.min(
        jnp.where(dist == minval, cols, CODES), axis=1, keepdims=True
    )


def _sc_gather_body(table_hbm, idx_hbm, out_hbm, idx_v, rows_v, sem):
    wid = lax.axis_index("s") * NC + lax.axis_index("c")
    chunk_base = wid * NCHUNK
    row_base = wid * BPW
    pltpu.sync_copy(idx_hbm.at[pl.ds(chunk_base, NCHUNK)], idx_v)
    copies = [
        pltpu.async_copy(
            table_hbm.at[idx_v.at[j]],
            rows_v.at[pl.ds(j * CHUNK, CHUNK)],
            sem,
        )
        for j in range(NCHUNK)
    ]
    for cp in copies:
        cp.wait()
    pltpu.sync_copy(rows_v, out_hbm.at[pl.ds(row_base, BPW)])


def kernel(x, embeddings):
    flat = x.reshape(B, LATENT)
    idx = pl.pallas_call(
        _argmin_body,
        grid=(B // ROWS_PER_BLOCK,),
        in_specs=[
            pl.BlockSpec((ROWS_PER_BLOCK, LATENT), lambda i: (i, 0)),
            pl.BlockSpec((LATENT, CODES), lambda i: (0, 0)),
        ],
        out_specs=pl.BlockSpec((ROWS_PER_BLOCK, 1), lambda i: (i, 0)),
        out_shape=jax.ShapeDtypeStruct((B, 1), jnp.int32),
    )(flat, embeddings)

    table = embeddings.T
    idx2d = idx.reshape(B // CHUNK, CHUNK)

    gather = pl.kernel(
        _sc_gather_body,
        mesh=plsc.VectorSubcoreMesh(core_axis_name="c", subcore_axis_name="s"),
        out_type=jax.ShapeDtypeStruct((B, LATENT), jnp.float32),
        scratch_types=[
            pltpu.VMEM((NCHUNK, CHUNK), jnp.int32),
            pltpu.VMEM((BPW, LATENT), jnp.float32),
            pltpu.SemaphoreType.DMA,
        ],
        compiler_params=pltpu.CompilerParams(use_tc_tiling_on_sc=False),
    )
    quantized = gather(table, idx2d)
    return quantized.reshape(x.shape)


# speedup vs baseline: 1.1311x; 1.1311x over previous
import jax
import jax.numpy as jnp
from jax import lax
from jax.experimental import pallas as pl
from jax.experimental.pallas import tpu as pltpu
from jax.experimental.pallas import tpu_sc as plsc

BATCH = 16
SEQ = 1024
LATENT = 64
CODES = 1024
B = BATCH * SEQ

NC = 2
NS = 16
NW = NC * NS
BPW = B // NW
CHUNK = 128
NCHUNK = BPW // CHUNK


def _argmin_body(x_ref, e_ref, idx_ref):
    xb = x_ref[0]
    em = e_ref[...]
    sim = jnp.dot(xb, em, preferred_element_type=jnp.float32)
    x_sq = jnp.sum(xb * xb, axis=1, keepdims=True)
    e_sq = jnp.sum(em * em, axis=0, keepdims=True)
    dist = (x_sq + e_sq) - 2.0 * sim
    minval = jnp.min(dist, axis=1, keepdims=True)
    cols = lax.broadcasted_iota(jnp.int32, (1, CODES), 1).astype(jnp.float32)
    idx = jnp.min(jnp.where(dist == minval, cols, float(CODES)), axis=1)
    idx_ref[...] = idx.astype(jnp.int32).reshape(8, 128)


def _sc_gather_body(table_hbm, idx_hbm, out_hbm, idx_v, rows_v, sem):
    wid = lax.axis_index("s") * NC + lax.axis_index("c")
    chunk_base = wid * NCHUNK
    b = wid // 2
    q0 = (wid % 2) * BPW
    pltpu.sync_copy(idx_hbm.at[pl.ds(chunk_base, NCHUNK)], idx_v)
    copies = [
        pltpu.async_copy(
            table_hbm.at[idx_v.at[j]],
            rows_v.at[pl.ds(j * CHUNK, CHUNK)],
            sem,
        )
        for j in range(NCHUNK)
    ]
    for cp in copies:
        cp.wait()
    pltpu.sync_copy(rows_v, out_hbm.at[b, pl.ds(q0, BPW)])


def kernel(x, embeddings):
    idx2d = pl.pallas_call(
        _argmin_body,
        grid=(BATCH,),
        in_specs=[
            pl.BlockSpec((1, SEQ, LATENT), lambda i: (i, 0, 0)),
            pl.BlockSpec((LATENT, CODES), lambda i: (0, 0)),
        ],
        out_specs=pl.BlockSpec((8, 128), lambda i: (i, 0)),
        out_shape=jax.ShapeDtypeStruct((B // CHUNK, CHUNK), jnp.int32),
    )(x, embeddings)

    table = embeddings.T

    gather = pl.kernel(
        _sc_gather_body,
        mesh=plsc.VectorSubcoreMesh(core_axis_name="c", subcore_axis_name="s"),
        out_type=jax.ShapeDtypeStruct((BATCH, SEQ, LATENT), jnp.float32),
        scratch_types=[
            pltpu.VMEM((NCHUNK, CHUNK), jnp.int32),
            pltpu.VMEM((BPW, LATENT), jnp.float32),
            pltpu.SemaphoreType.DMA,
        ],
        compiler_params=pltpu.CompilerParams(use_tc_tiling_on_sc=False),
    )
    return gather(table, idx2d)


# speedup vs baseline: 1.1628x; 1.0280x over previous
import jax
import jax.numpy as jnp
from jax import lax
from jax.experimental import pallas as pl
from jax.experimental.pallas import tpu as pltpu
from jax.experimental.pallas import tpu_sc as plsc

BATCH = 16
SEQ = 1024
LATENT = 64
CODES = 1024
B = BATCH * SEQ

NC = 2
NS = 16
NW = NC * NS
BPW = B // NW
CHUNK = 128
NCHUNK = BPW // CHUNK


def _argmin_body(x_ref, e_ref, idx_ref):
    xt = x_ref[0]
    em = e_ref[...]
    sim = lax.dot_general(
        xt, em, (((0,), (0,)), ((), ())), preferred_element_type=jnp.float32
    )
    x_sq = jnp.sum(xt * xt, axis=0, keepdims=True).T
    e_sq = jnp.sum(em * em, axis=0, keepdims=True)
    dist = (x_sq + e_sq) - 2.0 * sim
    minval = jnp.min(dist, axis=1, keepdims=True)
    cols = lax.broadcasted_iota(jnp.int32, (1, CODES), 1).astype(jnp.float32)
    idx = jnp.min(jnp.where(dist == minval, cols, float(CODES)), axis=1)
    idx_ref[...] = idx.astype(jnp.int32).reshape(8, 128)


def _sc_gather_body(table_hbm, idx_hbm, out_hbm, idx_v, rows_v, sem):
    wid = lax.axis_index("s") * NC + lax.axis_index("c")
    chunk_base = wid * NCHUNK
    b = wid // 2
    q0 = (wid % 2) * BPW
    pltpu.sync_copy(idx_hbm.at[pl.ds(chunk_base, NCHUNK)], idx_v)
    copies = [
        pltpu.async_copy(
            table_hbm.at[idx_v.at[j]],
            rows_v.at[pl.ds(j * CHUNK, CHUNK)],
            sem,
        )
        for j in range(NCHUNK)
    ]
    for cp in copies:
        cp.wait()
    pltpu.sync_copy(rows_v, out_hbm.at[b, pl.ds(q0, BPW)])


def kernel(x, embeddings):
    xt = jnp.swapaxes(x, 1, 2)
    idx2d = pl.pallas_call(
        _argmin_body,
        grid=(BATCH,),
        in_specs=[
            pl.BlockSpec((1, LATENT, SEQ), lambda i: (i, 0, 0)),
            pl.BlockSpec((LATENT, CODES), lambda i: (0, 0)),
        ],
        out_specs=pl.BlockSpec((8, 128), lambda i: (i, 0)),
        out_shape=jax.ShapeDtypeStruct((B // CHUNK, CHUNK), jnp.int32),
    )(xt, embeddings)

    table = embeddings.T

    gather = pl.kernel(
        _sc_gather_body,
        mesh=plsc.VectorSubcoreMesh(core_axis_name="c", subcore_axis_name="s"),
        out_type=jax.ShapeDtypeStruct((BATCH, SEQ, LATENT), jnp.float32),
        scratch_types=[
            pltpu.VMEM((NCHUNK, CHUNK), jnp.int32),
            pltpu.VMEM((BPW, LATENT), jnp.float32),
            pltpu.SemaphoreType.DMA,
        ],
        compiler_params=pltpu.CompilerParams(use_tc_tiling_on_sc=False),
    )
    return gather(table, idx2d)


# speedup vs baseline: 1.1905x; 1.0238x over previous
import jax
import jax.numpy as jnp
from jax import lax
from jax.experimental import pallas as pl
from jax.experimental.pallas import tpu as pltpu
from jax.experimental.pallas import tpu_sc as plsc

BATCH = 16
SEQ = 1024
LATENT = 64
CODES = 1024
B = BATCH * SEQ

NC = 2
NS = 16
NW = NC * NS
BPW = B // NW
CHUNK = 128
NCHUNK = BPW // CHUNK


def _argmin_body(x_ref, e_ref, idx_ref):
    xt = x_ref[0]
    em = e_ref[...]
    sim = lax.dot_general(
        xt, em, (((0,), (0,)), ((), ())), preferred_element_type=jnp.float32
    )
    x_sq = jnp.sum(xt * xt, axis=0, keepdims=True).T
    e_sq = jnp.sum(em * em, axis=0, keepdims=True)
    dist = (x_sq + e_sq) - 2.0 * sim
    minval = jnp.min(dist, axis=1, keepdims=True)
    cols = lax.broadcasted_iota(jnp.int32, (1, CODES), 1).astype(jnp.float32)
    idx = jnp.min(jnp.where(dist == minval, cols, float(CODES)), axis=1)
    idx_ref[...] = idx.astype(jnp.int32).reshape(8, 128)


def _sc_gather_body(em_hbm, idx_hbm, out_hbm, em_v, idx_v, tile_v):
    wid = lax.axis_index("s") * NC + lax.axis_index("c")
    b = wid // 2
    qt0 = (wid % 2) * 4
    pltpu.sync_copy(em_hbm, em_v)
    pltpu.sync_copy(idx_hbm.at[pl.ds(wid * 4, 4)], idx_v)

    def slab(s, carry):
        r = s // 8
        qig = (s % 8) * 16
        idx16 = idx_v[r, pl.ds(qig, 16)]
        base16 = ((idx16 >> 7) << 10) + (idx16 & 127)
        for l in range(LATENT):
            lt, li = l // 8, l % 8
            g = plsc.load_gather(em_v, [base16 + (lt * 8192 + li * 128)])
            tile_v[lt, r, li, pl.ds(qig, 16)] = g
        return carry

    lax.fori_loop(0, 32, slab, 0)

    for lt in range(8):
        pltpu.sync_copy(tile_v.at[lt], out_hbm.at[b, lt, pl.ds(qt0, 4)])


def kernel(x, embeddings):
    xt = jnp.swapaxes(x, 1, 2)
    idx2d = pl.pallas_call(
        _argmin_body,
        grid=(BATCH,),
        in_specs=[
            pl.BlockSpec((1, LATENT, SEQ), lambda i: (i, 0, 0)),
            pl.BlockSpec((LATENT, CODES), lambda i: (0, 0)),
        ],
        out_specs=pl.BlockSpec((8, 128), lambda i: (i, 0)),
        out_shape=jax.ShapeDtypeStruct((B // CHUNK, CHUNK), jnp.int32),
    )(xt, embeddings)

    em_flat = embeddings.reshape(8, 8, 8, 128).transpose(0, 2, 1, 3).reshape(-1)

    gather = pl.kernel(
        _sc_gather_body,
        mesh=plsc.VectorSubcoreMesh(core_axis_name="c", subcore_axis_name="s"),
        out_type=jax.ShapeDtypeStruct((BATCH, 8, 8, 8, 128), jnp.float32),
        scratch_types=[
            pltpu.VMEM((LATENT * CODES,), jnp.float32),
            pltpu.VMEM((4, CHUNK), jnp.int32),
            pltpu.VMEM((8, 4, 8, 128), jnp.float32),
        ],
        compiler_params=pltpu.CompilerParams(
            use_tc_tiling_on_sc=False, needs_layout_passes=False
        ),
    )
    out5 = gather(em_flat, idx2d)
    return out5.transpose(0, 2, 4, 1, 3).reshape(BATCH, SEQ, LATENT)


# speedup vs baseline: 1.2803x; 1.0755x over previous
import jax
import jax.numpy as jnp
from jax import lax
from jax.experimental import pallas as pl
from jax.experimental.pallas import tpu as pltpu
from jax.experimental.pallas import tpu_sc as plsc

BATCH = 16
SEQ = 1024
LATENT = 64
CODES = 1024
B = BATCH * SEQ

NC = 2
NS = 16
NW = NC * NS
BPW = B // NW
CHUNK = 128
NCHUNK = BPW // CHUNK


def _argmin_body(x_ref, e_ref, idx_ref):
    xt = x_ref[0]
    em = e_ref[...]
    sim = lax.dot_general(
        xt, em, (((0,), (0,)), ((), ())), preferred_element_type=jnp.float32
    )
    x_sq = jnp.sum(xt * xt, axis=0, keepdims=True).T
    e_sq = jnp.sum(em * em, axis=0, keepdims=True)
    dist = (x_sq + e_sq) - 2.0 * sim
    minval = jnp.min(dist, axis=1, keepdims=True)
    cols = lax.broadcasted_iota(jnp.int32, (1, CODES), 1).astype(jnp.float32)
    idx = jnp.min(jnp.where(dist == minval, cols, float(CODES)), axis=1)
    idx_ref[...] = idx.astype(jnp.int32).reshape(8, 128)


def _sc_gather_body(em_hbm, idx_hbm, out_hbm, em_v, idx_v, tile_v):
    wid = lax.axis_index("s") * NC + lax.axis_index("c")
    b = wid // 2
    qt0 = (wid % 2) * 4
    pltpu.sync_copy(em_hbm, em_v)
    pltpu.sync_copy(idx_hbm.at[pl.ds(wid * 4, 4)], idx_v)

    @plsc.parallel_loop(0, 32, unroll=2)
    def _(s):
        r = s // 8
        qig = (s % 8) * 16
        idx16 = idx_v[r, pl.ds(qig, 16)]
        base16 = ((idx16 >> 7) << 10) + (idx16 & 127)
        for l in range(LATENT):
            lt, li = l // 8, l % 8
            g = plsc.load_gather(em_v, [base16 + (lt * 8192 + li * 128)])
            tile_v[lt, r, li, pl.ds(qig, 16)] = g

    for lt in range(8):
        pltpu.sync_copy(tile_v.at[lt], out_hbm.at[b, lt, pl.ds(qt0, 4)])


def kernel(x, embeddings):
    xt = jnp.swapaxes(x, 1, 2)
    idx2d = pl.pallas_call(
        _argmin_body,
        grid=(BATCH,),
        in_specs=[
            pl.BlockSpec((1, LATENT, SEQ), lambda i: (i, 0, 0)),
            pl.BlockSpec((LATENT, CODES), lambda i: (0, 0)),
        ],
        out_specs=pl.BlockSpec((8, 128), lambda i: (i, 0)),
        out_shape=jax.ShapeDtypeStruct((B // CHUNK, CHUNK), jnp.int32),
    )(xt, embeddings)

    em_flat = embeddings.reshape(8, 8, 8, 128).transpose(0, 2, 1, 3).reshape(-1)

    gather = pl.kernel(
        _sc_gather_body,
        mesh=plsc.VectorSubcoreMesh(core_axis_name="c", subcore_axis_name="s"),
        out_type=jax.ShapeDtypeStruct((BATCH, 8, 8, 8, 128), jnp.float32),
        scratch_types=[
            pltpu.VMEM((LATENT * CODES,), jnp.float32),
            pltpu.VMEM((4, CHUNK), jnp.int32),
            pltpu.VMEM((8, 4, 8, 128), jnp.float32),
        ],
        compiler_params=pltpu.CompilerParams(
            use_tc_tiling_on_sc=False, needs_layout_passes=False
        ),
    )
    out5 = gather(em_flat, idx2d)
    return out5.transpose(0, 2, 4, 1, 3).reshape(BATCH, SEQ, LATENT)


# speedup vs baseline: 1.3092x; 1.0225x over previous
import jax
import jax.numpy as jnp
from jax import lax
from jax.experimental import pallas as pl
from jax.experimental.pallas import tpu as pltpu
from jax.experimental.pallas import tpu_sc as plsc

BATCH = 16
SEQ = 1024
LATENT = 64
CODES = 1024
B = BATCH * SEQ

NC = 2
NS = 16
NW = NC * NS
BPW = B // NW
CHUNK = 128
NCHUNK = BPW // CHUNK


def _argmin_body(x_ref, e_ref, idx_ref):
    xt = x_ref[0]
    em = e_ref[...]
    sim = lax.dot_general(
        xt, em, (((0,), (0,)), ((), ())), preferred_element_type=jnp.float32
    )
    x_sq = jnp.sum(xt * xt, axis=0, keepdims=True).T
    e_sq = jnp.sum(em * em, axis=0, keepdims=True)
    dist = (x_sq + e_sq) - 2.0 * sim
    minval = jnp.min(dist, axis=1, keepdims=True)
    cols = lax.broadcasted_iota(jnp.int32, (1, CODES), 1).astype(jnp.float32)
    idx = jnp.min(jnp.where(dist == minval, cols, float(CODES)), axis=1)
    idx_ref[...] = idx.astype(jnp.int32).reshape(8, 128)


def _sc_gather_body(em_hbm, idx_hbm, out_hbm, em_v, idx_v, tile_v):
    wid = lax.axis_index("s") * NC + lax.axis_index("c")
    b = wid // 2
    qt0 = (wid % 2) * 4
    pltpu.sync_copy(em_hbm, em_v)
    pltpu.sync_copy(idx_hbm.at[pl.ds(wid * 4, 4)], idx_v)

    @plsc.parallel_loop(0, 32, unroll=4)
    def _(s):
        r = s // 8
        qig = (s % 8) * 16
        idx16 = idx_v[r, pl.ds(qig, 16)]
        base16 = ((idx16 >> 7) << 10) + (idx16 & 127)
        for l in range(LATENT):
            lt, li = l // 8, l % 8
            g = plsc.load_gather(em_v, [base16 + (lt * 8192 + li * 128)])
            tile_v[lt, r, li, pl.ds(qig, 16)] = g

    for lt in range(8):
        pltpu.sync_copy(tile_v.at[lt], out_hbm.at[b, lt, pl.ds(qt0, 4)])


def kernel(x, embeddings):
    xt = jnp.swapaxes(x, 1, 2)
    idx2d = pl.pallas_call(
        _argmin_body,
        grid=(BATCH,),
        in_specs=[
            pl.BlockSpec((1, LATENT, SEQ), lambda i: (i, 0, 0)),
            pl.BlockSpec((LATENT, CODES), lambda i: (0, 0)),
        ],
        out_specs=pl.BlockSpec((8, 128), lambda i: (i, 0)),
        out_shape=jax.ShapeDtypeStruct((B // CHUNK, CHUNK), jnp.int32),
    )(xt, embeddings)

    em_flat = embeddings.reshape(8, 8, 8, 128).transpose(0, 2, 1, 3).reshape(-1)

    gather = pl.kernel(
        _sc_gather_body,
        mesh=plsc.VectorSubcoreMesh(core_axis_name="c", subcore_axis_name="s"),
        out_type=jax.ShapeDtypeStruct((BATCH, 8, 8, 8, 128), jnp.float32),
        scratch_types=[
            pltpu.VMEM((LATENT * CODES,), jnp.float32),
            pltpu.VMEM((4, CHUNK), jnp.int32),
            pltpu.VMEM((8, 4, 8, 128), jnp.float32),
        ],
        compiler_params=pltpu.CompilerParams(
            use_tc_tiling_on_sc=False, needs_layout_passes=False
        ),
    )
    out5 = gather(em_flat, idx2d)
    return out5.transpose(0, 2, 4, 1, 3).reshape(BATCH, SEQ, LATENT)


# speedup vs baseline: 1.3101x; 1.0007x over previous
import jax
import jax.numpy as jnp
from jax import lax
from jax.experimental import pallas as pl
from jax.experimental.pallas import tpu as pltpu
from jax.experimental.pallas import tpu_sc as plsc

BATCH = 16
SEQ = 1024
LATENT = 64
CODES = 1024
B = BATCH * SEQ

NC = 2
NS = 16
NW = NC * NS
BPW = B // NW
CHUNK = 128
NCHUNK = BPW // CHUNK


def _argmin_body(x_ref, e_ref, idx_ref):
    xt = x_ref[0]
    em = e_ref[...]
    sim = lax.dot_general(
        xt, em, (((0,), (0,)), ((), ())), preferred_element_type=jnp.float32
    )
    x_sq = jnp.sum(xt * xt, axis=0, keepdims=True).T
    e_sq = jnp.sum(em * em, axis=0, keepdims=True)
    dist = (x_sq + e_sq) - 2.0 * sim
    minval = jnp.min(dist, axis=1, keepdims=True)
    cols = lax.broadcasted_iota(jnp.int32, (1, CODES), 1).astype(jnp.float32)
    idx = jnp.min(jnp.where(dist == minval, cols, float(CODES)), axis=1)
    idx_ref[...] = idx.astype(jnp.int32).reshape(8, 128)


def _sc_gather_body(em_hbm, idx_hbm, out_hbm, em_v, idx_v, tile_v):
    wid = lax.axis_index("s") * NC + lax.axis_index("c")
    b = wid // 2
    qt0 = (wid % 2) * 4
    pltpu.sync_copy(em_hbm, em_v)
    pltpu.sync_copy(idx_hbm.at[pl.ds(wid * 4, 4)], idx_v)

    @plsc.parallel_loop(0, 32, unroll=4)
    def _(s):
        r = s // 8
        qig = (s % 8) * 16
        idx16 = idx_v[r, pl.ds(qig, 16)]
        base16 = ((idx16 >> 7) << 10) + (idx16 & 127)
        for g0 in range(0, LATENT, 16):
            gs = [
                plsc.load_gather(
                    em_v, [base16 + ((l // 8) * 8192 + (l % 8) * 128)]
                )
                for l in range(g0, g0 + 16)
            ]
            for k, l in enumerate(range(g0, g0 + 16)):
                tile_v[l // 8, r, l % 8, pl.ds(qig, 16)] = gs[k]

    for lt in range(8):
        pltpu.sync_copy(tile_v.at[lt], out_hbm.at[b, lt, pl.ds(qt0, 4)])


def kernel(x, embeddings):
    xt = jnp.swapaxes(x, 1, 2)
    idx2d = pl.pallas_call(
        _argmin_body,
        grid=(BATCH,),
        in_specs=[
            pl.BlockSpec((1, LATENT, SEQ), lambda i: (i, 0, 0)),
            pl.BlockSpec((LATENT, CODES), lambda i: (0, 0)),
        ],
        out_specs=pl.BlockSpec((8, 128), lambda i: (i, 0)),
        out_shape=jax.ShapeDtypeStruct((B // CHUNK, CHUNK), jnp.int32),
    )(xt, embeddings)

    em_flat = embeddings.reshape(8, 8, 8, 128).transpose(0, 2, 1, 3).reshape(-1)

    gather = pl.kernel(
        _sc_gather_body,
        mesh=plsc.VectorSubcoreMesh(core_axis_name="c", subcore_axis_name="s"),
        out_type=jax.ShapeDtypeStruct((BATCH, 8, 8, 8, 128), jnp.float32),
        scratch_types=[
            pltpu.VMEM((LATENT * CODES,), jnp.float32),
            pltpu.VMEM((4, CHUNK), jnp.int32),
            pltpu.VMEM((8, 4, 8, 128), jnp.float32),
        ],
        compiler_params=pltpu.CompilerParams(
            use_tc_tiling_on_sc=False, needs_layout_passes=False
        ),
    )
    out5 = gather(em_flat, idx2d)
    return out5.transpose(0, 2, 4, 1, 3).reshape(BATCH, SEQ, LATENT)
